# Initial kernel scaffold; baseline (speedup 1.0000x reference)
import jax
import jax.numpy as jnp
from jax.experimental import pallas as pl


def kernel(feat_src, feat_dst, edge_index, dst_trans_weight, src_trans_weight, rel_emb, rel_trans_weight):
    raise NotImplementedError("write your pallas kernel here")


# speedup vs baseline: 41.3346x; 41.3346x over previous
import functools

import jax
import jax.numpy as jnp
from jax import lax
from jax.experimental import pallas as pl
from jax.experimental.pallas import tpu as pltpu
from jax.experimental.pallas import tpu_sc as plsc

N = 10000
E = 320000
D_IN = 128
H = 8
D = 16
HD = H * D

NC = 2
NS = 16
NW = NC * NS
EPT = E // NW
CH = 80
NCHUNK = EPT // CH
NP = 10240
RT = NP // NS
ZR = 128

_mesh = plsc.VectorSubcoreMesh(core_axis_name="c", subcore_axis_name="s")


def _tc1_body(fsrc_ref, fdst_ref, w_ref, wes_ref, wed_ref,
              fs_ref, es_ref, ed_ref, mes_ref, med_ref):
    i = pl.program_id(0)
    fs_ref[...] = jnp.dot(fsrc_ref[...], w_ref[...],
                          preferred_element_type=jnp.float32)
    es = jnp.dot(fsrc_ref[...], wes_ref[...],
                 preferred_element_type=jnp.float32)
    ed = jnp.dot(fdst_ref[...], wed_ref[...],
                 preferred_element_type=jnp.float32)
    es_ref[...] = es
    ed_ref[...] = ed
    mes = jnp.max(es, axis=0, keepdims=True)
    med = jnp.max(ed, axis=0, keepdims=True)

    @pl.when(i == 0)
    def _():
        mes_ref[...] = mes
        med_ref[...] = med

    @pl.when(i > 0)
    def _():
        mes_ref[...] = jnp.maximum(mes_ref[...], mes)
        med_ref[...] = jnp.maximum(med_ref[...], med)


def _tc1(feat_src, feat_dst, w_src, w_es16, w_ed16):
    blk = 1000
    grid = N // blk
    return pl.pallas_call(
        _tc1_body,
        grid=(grid,),
        in_specs=[
            pl.BlockSpec((blk, D_IN), lambda i: (i, 0)),
            pl.BlockSpec((blk, D_IN), lambda i: (i, 0)),
            pl.BlockSpec((D_IN, HD), lambda i: (0, 0)),
            pl.BlockSpec((D_IN, 16), lambda i: (0, 0)),
            pl.BlockSpec((D_IN, 16), lambda i: (0, 0)),
        ],
        out_specs=[
            pl.BlockSpec((blk, HD), lambda i: (i, 0)),
            pl.BlockSpec((blk, 16), lambda i: (i, 0)),
            pl.BlockSpec((blk, 16), lambda i: (i, 0)),
            pl.BlockSpec((1, 16), lambda i: (0, 0)),
            pl.BlockSpec((1, 16), lambda i: (0, 0)),
        ],
        out_shape=[
            jax.ShapeDtypeStruct((N, HD), jnp.float32),
            jax.ShapeDtypeStruct((N, 16), jnp.float32),
            jax.ShapeDtypeStruct((N, 16), jnp.float32),
            jax.ShapeDtypeStruct((1, 16), jnp.float32),
            jax.ShapeDtypeStruct((1, 16), jnp.float32),
        ],
    )(feat_src, feat_dst, w_src, w_es16, w_ed16)


@functools.partial(
    pl.kernel,
    out_type=[
        jax.ShapeDtypeStruct((E, 16), jnp.float32),
        jax.ShapeDtypeStruct((NC * NP, 16), jnp.float32),
    ],
    mesh=_mesh,
    compiler_params=pltpu.CompilerParams(use_tc_tiling_on_sc=False),
    scratch_types=[
        pltpu.VMEM((CH,), jnp.int32),
        pltpu.VMEM((CH,), jnp.int32),
        pltpu.VMEM((CH, 16), jnp.float32),
        pltpu.VMEM((CH, 16), jnp.float32),
        pltpu.VMEM((CH, 16), jnp.float32),
        pltpu.VMEM((16,), jnp.float32),
        pltpu.VMEM((RT, 16), jnp.float32),
        pltpu.VMEM_SHARED((NP, 16), jnp.float32),
        pltpu.SemaphoreType.DMA,
        pltpu.SemaphoreType.DMA,
    ],
)
def _sc_a(src_h, dst_h, es_h, ed_h, c_h, p_h, sp_h,
          sidx, didx, esb, edb, pb, cb, zb, s_sh, sem1, sem2):
    cid = lax.axis_index("c")
    sid = lax.axis_index("s")
    wid = cid * NS + sid

    def _zrow(i, carry):
        zb[i] = jnp.zeros((16,), jnp.float32)
        return carry
    lax.fori_loop(0, RT, _zrow, 0)
    pltpu.sync_copy(zb, s_sh.at[pl.ds(sid * RT, RT)])
    plsc.subcore_barrier()

    pltpu.sync_copy(c_h, cb)
    cv = cb[...]

    base0 = wid * EPT

    def _chunk(j, carry):
        base = base0 + j * CH
        pltpu.sync_copy(src_h.at[pl.ds(base, CH)], sidx)
        pltpu.sync_copy(dst_h.at[pl.ds(base, CH)], didx)
        pltpu.async_copy(es_h.at[sidx], esb, sem1).wait()
        pltpu.async_copy(ed_h.at[didx], edb, sem2).wait()

        U = 8

        def _inner(k, c2):
            for u in range(U):
                kk = k * U + u
                e = esb[kk] + edb[kk]
                e = jnp.where(e > 0, e, e * 0.01)
                pb[kk] = jnp.exp(e - cv)
            return c2
        lax.fori_loop(0, CH // U, _inner, 0)

        pltpu.sync_copy(pb, p_h.at[pl.ds(base, CH)])
        pltpu.sync_copy(pb, s_sh.at[didx], add=True)
        return carry
    lax.fori_loop(0, NCHUNK, _chunk, 0)

    plsc.subcore_barrier()
    pltpu.sync_copy(s_sh.at[pl.ds(sid * RT, RT)],
                    sp_h.at[pl.ds(cid * NP + sid * RT, RT)])


def _tc2_body(sp_ref, rinv_ref):
    s = sp_ref[0:NP, :] + sp_ref[NP:2 * NP, :]
    rinv_ref[...] = 1.0 / s


def _tc2(sp):
    return pl.pallas_call(
        _tc2_body,
        out_shape=jax.ShapeDtypeStruct((NP, 16), jnp.float32),
    )(sp)


@functools.partial(
    pl.kernel,
    out_type=jax.ShapeDtypeStruct((NC * NP, HD), jnp.float32),
    mesh=_mesh,
    compiler_params=pltpu.CompilerParams(use_tc_tiling_on_sc=False),
    scratch_types=[
        pltpu.VMEM((CH,), jnp.int32),
        pltpu.VMEM((CH,), jnp.int32),
        pltpu.VMEM((CH, HD), jnp.float32),
        pltpu.VMEM((CH, 16), jnp.float32),
        pltpu.VMEM((CH, 16), jnp.float32),
        pltpu.VMEM((ZR, HD), jnp.float32),
        pltpu.VMEM_SHARED((NP, HD), jnp.float32),
        pltpu.SemaphoreType.DMA,
        pltpu.SemaphoreType.DMA,
    ],
)
def _sc_c(src_h, dst_h, fs_h, rv_h, p_h, op_h,
          sidx, didx, fsb, rvb, pb, zb, o_sh, sem1, sem2):
    cid = lax.axis_index("c")
    sid = lax.axis_index("s")
    wid = cid * NS + sid

    def _zrow(i, carry):
        for l in range(HD // 16):
            zb[i, pl.ds(l * 16, 16)] = jnp.zeros((16,), jnp.float32)
        return carry
    lax.fori_loop(0, ZR, _zrow, 0)
    for rep in range(RT // ZR):
        pltpu.sync_copy(zb, o_sh.at[pl.ds(sid * RT + rep * ZR, ZR)])
    plsc.subcore_barrier()

    base0 = wid * EPT

    def _chunk(j, carry):
        base = base0 + j * CH
        pltpu.sync_copy(src_h.at[pl.ds(base, CH)], sidx)
        pltpu.sync_copy(dst_h.at[pl.ds(base, CH)], didx)
        pltpu.async_copy(fs_h.at[sidx], fsb, sem1).wait()
        pltpu.async_copy(rv_h.at[didx], rvb, sem2).wait()
        pltpu.sync_copy(p_h.at[pl.ds(base, CH)], pb)

        def _edge(k, c2):
            av = pb[k] * rvb[k]
            for h in range(H):
                a = av[h]
                fsb[k, pl.ds(h * D, D)] = fsb[k, pl.ds(h * D, D)] * a
            return c2
        lax.fori_loop(0, CH, _edge, 0)

        pltpu.sync_copy(fsb, o_sh.at[didx], add=True)
        return carry
    lax.fori_loop(0, NCHUNK, _chunk, 0)

    plsc.subcore_barrier()
    for rep in range(RT // ZR):
        pltpu.sync_copy(o_sh.at[pl.ds(sid * RT + rep * ZR, ZR)],
                        op_h.at[pl.ds(cid * NP + sid * RT + rep * ZR, ZR)])


def _tc3_body(o0_ref, o1_ref, out_ref):
    out_ref[...] = jnp.maximum(o0_ref[...] + o1_ref[...], 0.0)


def _tc3(op):
    blk = 1024
    grid = NP // blk
    return pl.pallas_call(
        _tc3_body,
        grid=(grid,),
        in_specs=[
            pl.BlockSpec((blk, HD), lambda i: (i, 0)),
            pl.BlockSpec((blk, HD), lambda i: (NP // blk + i, 0)),
        ],
        out_specs=pl.BlockSpec((blk, HD), lambda i: (i, 0)),
        out_shape=jax.ShapeDtypeStruct((NP, HD), jnp.float32),
    )(op, op)


def kernel(feat_src, feat_dst, edge_index, dst_trans_weight, src_trans_weight,
           rel_emb, rel_trans_weight):
    ei = edge_index.astype(jnp.int32)
    src = ei[0]
    dst = ei[1]

    rel_attn = jnp.matmul(rel_emb[None, :], rel_trans_weight).reshape(H, 2 * D)
    w_es = (src_trans_weight.reshape(D_IN, H, D)
            * rel_attn[:, D:][None]).sum(-1)
    w_ed = (dst_trans_weight.reshape(D_IN, H, D)
            * rel_attn[:, :D][None]).sum(-1)
    pad = jnp.zeros((D_IN, 16 - H), jnp.float32)
    w_es16 = jnp.concatenate([w_es, pad], axis=1)
    w_ed16 = jnp.concatenate([w_ed, pad], axis=1)

    fs, es16, ed16, mes, med = _tc1(feat_src, feat_dst, src_trans_weight,
                                    w_es16, w_ed16)
    c16 = jnp.maximum(mes + med, 0.0).reshape(16)

    p16, sp = _sc_a(src, dst, es16, ed16, c16)
    rinv16 = _tc2(sp)
    op = _sc_c(src, dst, fs, rinv16, p16)
    return _tc3(op)[:N]


# speedup vs baseline: 60.8952x; 1.4732x over previous
import functools

import jax
import jax.numpy as jnp
from jax import lax
from jax.experimental import pallas as pl
from jax.experimental.pallas import tpu as pltpu
from jax.experimental.pallas import tpu_sc as plsc

N = 10000
E = 320000
D_IN = 128
H = 8
D = 16
HD = H * D

NC = 2
NS = 16
NW = NC * NS
EPT = E // NW
CH = 80
NCHUNK = EPT // CH
NP = 10240
RT = NP // NS
ZR = 128

_mesh = plsc.VectorSubcoreMesh(core_axis_name="c", subcore_axis_name="s")


def _tc1_body(fsrc_ref, fdst_ref, w_ref, wes_ref, wed_ref,
              fs_ref, es_ref, ed_ref, mes_ref, med_ref):
    i = pl.program_id(0)
    fs_ref[...] = jnp.dot(fsrc_ref[...], w_ref[...],
                          preferred_element_type=jnp.float32)
    es = jnp.dot(fsrc_ref[...], wes_ref[...],
                 preferred_element_type=jnp.float32)
    ed = jnp.dot(fdst_ref[...], wed_ref[...],
                 preferred_element_type=jnp.float32)
    es_ref[...] = es
    ed_ref[...] = ed
    mes = jnp.max(es, axis=0, keepdims=True)
    med = jnp.max(ed, axis=0, keepdims=True)

    @pl.when(i == 0)
    def _():
        mes_ref[...] = mes
        med_ref[...] = med

    @pl.when(i > 0)
    def _():
        mes_ref[...] = jnp.maximum(mes_ref[...], mes)
        med_ref[...] = jnp.maximum(med_ref[...], med)


def _tc1(feat_src, feat_dst, w_src, w_es16, w_ed16):
    blk = 1000
    grid = N // blk
    return pl.pallas_call(
        _tc1_body,
        grid=(grid,),
        in_specs=[
            pl.BlockSpec((blk, D_IN), lambda i: (i, 0)),
            pl.BlockSpec((blk, D_IN), lambda i: (i, 0)),
            pl.BlockSpec((D_IN, HD), lambda i: (0, 0)),
            pl.BlockSpec((D_IN, 16), lambda i: (0, 0)),
            pl.BlockSpec((D_IN, 16), lambda i: (0, 0)),
        ],
        out_specs=[
            pl.BlockSpec((blk, HD), lambda i: (i, 0)),
            pl.BlockSpec((blk, 16), lambda i: (i, 0)),
            pl.BlockSpec((blk, 16), lambda i: (i, 0)),
            pl.BlockSpec((1, 16), lambda i: (0, 0)),
            pl.BlockSpec((1, 16), lambda i: (0, 0)),
        ],
        out_shape=[
            jax.ShapeDtypeStruct((N, HD), jnp.float32),
            jax.ShapeDtypeStruct((N, 16), jnp.float32),
            jax.ShapeDtypeStruct((N, 16), jnp.float32),
            jax.ShapeDtypeStruct((1, 16), jnp.float32),
            jax.ShapeDtypeStruct((1, 16), jnp.float32),
        ],
    )(feat_src, feat_dst, w_src, w_es16, w_ed16)


@functools.partial(
    pl.kernel,
    out_type=[
        jax.ShapeDtypeStruct((E, 16), jnp.float32),
        jax.ShapeDtypeStruct((NC * NP, 16), jnp.float32),
    ],
    mesh=_mesh,
    compiler_params=pltpu.CompilerParams(use_tc_tiling_on_sc=False),
    scratch_types=[
        pltpu.VMEM((2, CH), jnp.int32),
        pltpu.VMEM((CH, 16), jnp.float32),
        pltpu.VMEM((CH, 16), jnp.float32),
        pltpu.VMEM((CH, 16), jnp.float32),
        pltpu.VMEM((16,), jnp.float32),
        pltpu.VMEM((RT, 16), jnp.float32),
        pltpu.VMEM_SHARED((NP, 16), jnp.float32),
        pltpu.SemaphoreType.DMA,
        pltpu.SemaphoreType.DMA,
        pltpu.SemaphoreType.DMA,
        pltpu.SemaphoreType.DMA,
    ],
)
def _sc_a(ei2_h, es_h, ed_h, c_h, p_h, sp_h,
          sdb, esb, edb, pb, cb, zb, s_sh, sem1, sem2, sem3, sem4):
    cid = lax.axis_index("c")
    sid = lax.axis_index("s")
    wid = cid * NS + sid

    def _zrow(i, carry):
        zb[i] = jnp.zeros((16,), jnp.float32)
        return carry
    lax.fori_loop(0, RT, _zrow, 0)
    pltpu.sync_copy(zb, s_sh.at[pl.ds(sid * RT, RT)])
    plsc.subcore_barrier()

    pltpu.sync_copy(c_h, cb)
    cv = cb[...]

    base0 = wid * EPT

    def _chunk(j, carry):
        base = base0 + j * CH
        g = wid * NCHUNK + j
        pltpu.sync_copy(ei2_h.at[pl.ds(2 * g, 2)], sdb)
        d1 = pltpu.async_copy(es_h.at[sdb.at[0]], esb, sem1)
        d2 = pltpu.async_copy(ed_h.at[sdb.at[1]], edb, sem2)
        d1.wait()
        d2.wait()

        U = 8

        def _inner(k, c2):
            for u in range(U):
                kk = k * U + u
                e = esb[kk] + edb[kk]
                e = jnp.where(e > 0, e, e * 0.01)
                pb[kk] = jnp.exp(e - cv)
            return c2
        lax.fori_loop(0, CH // U, _inner, 0)

        d3 = pltpu.async_copy(pb, p_h.at[pl.ds(base, CH)], sem3)
        d4 = pltpu.async_copy(pb, s_sh.at[sdb.at[1]], sem4, add=True)
        d3.wait()
        d4.wait()
        return carry
    lax.fori_loop(0, NCHUNK, _chunk, 0)

    plsc.subcore_barrier()
    pltpu.sync_copy(s_sh.at[pl.ds(sid * RT, RT)],
                    sp_h.at[pl.ds(cid * NP + sid * RT, RT)])


def _tc2_body(sp_ref, rinv_ref):
    s = sp_ref[0:NP, :] + sp_ref[NP:2 * NP, :]
    rinv_ref[...] = 1.0 / s


def _tc2(sp):
    return pl.pallas_call(
        _tc2_body,
        out_shape=jax.ShapeDtypeStruct((NP, 16), jnp.float32),
    )(sp)


@functools.partial(
    pl.kernel,
    out_type=jax.ShapeDtypeStruct((NC * NP, HD), jnp.float32),
    mesh=_mesh,
    compiler_params=pltpu.CompilerParams(use_tc_tiling_on_sc=False),
    scratch_types=[
        pltpu.VMEM((2, CH), jnp.int32),
        pltpu.VMEM((CH, HD), jnp.float32),
        pltpu.VMEM((CH, 16), jnp.float32),
        pltpu.VMEM((CH, 16), jnp.float32),
        pltpu.VMEM((ZR, HD), jnp.float32),
        pltpu.VMEM_SHARED((NP, HD), jnp.float32),
        pltpu.SemaphoreType.DMA,
        pltpu.SemaphoreType.DMA,
        pltpu.SemaphoreType.DMA,
    ],
)
def _sc_c(ei2_h, fs_h, rv_h, p_h, op_h,
          sdb, fsb, rvb, pb, zb, o_sh, sem1, sem2, sem3):
    cid = lax.axis_index("c")
    sid = lax.axis_index("s")
    wid = cid * NS + sid

    def _zrow(i, carry):
        for l in range(HD // 16):
            zb[i, pl.ds(l * 16, 16)] = jnp.zeros((16,), jnp.float32)
        return carry
    lax.fori_loop(0, ZR, _zrow, 0)
    for rep in range(RT // ZR):
        pltpu.sync_copy(zb, o_sh.at[pl.ds(sid * RT + rep * ZR, ZR)])
    plsc.subcore_barrier()

    base0 = wid * EPT

    def _chunk(j, carry):
        base = base0 + j * CH
        g = wid * NCHUNK + j
        pltpu.sync_copy(ei2_h.at[pl.ds(2 * g, 2)], sdb)
        d1 = pltpu.async_copy(fs_h.at[sdb.at[0]], fsb, sem1)
        d2 = pltpu.async_copy(rv_h.at[sdb.at[1]], rvb, sem2)
        d3 = pltpu.async_copy(p_h.at[pl.ds(base, CH)], pb, sem3)
        d1.wait()
        d2.wait()
        d3.wait()

        def _edge(k, c2):
            av = pb[k] * rvb[k]
            for h in range(H):
                a = av[h]
                fsb[k, pl.ds(h * D, D)] = fsb[k, pl.ds(h * D, D)] * a
            return c2
        lax.fori_loop(0, CH, _edge, 0)

        pltpu.sync_copy(fsb, o_sh.at[sdb.at[1]], add=True)
        return carry
    lax.fori_loop(0, NCHUNK, _chunk, 0)

    plsc.subcore_barrier()
    for rep in range(RT // ZR):
        pltpu.sync_copy(o_sh.at[pl.ds(sid * RT + rep * ZR, ZR)],
                        op_h.at[pl.ds(cid * NP + sid * RT + rep * ZR, ZR)])


def _tc3_body(o0_ref, o1_ref, out_ref):
    out_ref[...] = jnp.maximum(o0_ref[...] + o1_ref[...], 0.0)


def _tc3(op):
    blk = 1024
    grid = NP // blk
    return pl.pallas_call(
        _tc3_body,
        grid=(grid,),
        in_specs=[
            pl.BlockSpec((blk, HD), lambda i: (i, 0)),
            pl.BlockSpec((blk, HD), lambda i: (NP // blk + i, 0)),
        ],
        out_specs=pl.BlockSpec((blk, HD), lambda i: (i, 0)),
        out_shape=jax.ShapeDtypeStruct((NP, HD), jnp.float32),
    )(op, op)


def kernel(feat_src, feat_dst, edge_index, dst_trans_weight, src_trans_weight,
           rel_emb, rel_trans_weight):
    ei = edge_index.astype(jnp.int32)
    src2 = ei[0].reshape(E // CH, CH)
    dst2 = ei[1].reshape(E // CH, CH)
    ei2 = jnp.stack([src2, dst2], axis=1).reshape(2 * (E // CH), CH)

    rel_attn = jnp.matmul(rel_emb[None, :], rel_trans_weight).reshape(H, 2 * D)
    w_es = (src_trans_weight.reshape(D_IN, H, D)
            * rel_attn[:, D:][None]).sum(-1)
    w_ed = (dst_trans_weight.reshape(D_IN, H, D)
            * rel_attn[:, :D][None]).sum(-1)
    pad = jnp.zeros((D_IN, 16 - H), jnp.float32)
    w_es16 = jnp.concatenate([w_es, pad], axis=1)
    w_ed16 = jnp.concatenate([w_ed, pad], axis=1)

    fs, es16, ed16, mes, med = _tc1(feat_src, feat_dst, src_trans_weight,
                                    w_es16, w_ed16)
    c16 = jnp.maximum(mes + med, 0.0).reshape(16)

    p16, sp = _sc_a(ei2, es16, ed16, c16)
    rinv16 = _tc2(sp)
    op = _sc_c(ei2, fs, rinv16, p16)
    return _tc3(op)[:N]


# speedup vs baseline: 80.2235x; 1.3174x over previous
import functools

import jax
import jax.numpy as jnp
from jax import lax
from jax.experimental import pallas as pl
from jax.experimental.pallas import tpu as pltpu
from jax.experimental.pallas import tpu_sc as plsc

N = 10000
E = 320000
D_IN = 128
H = 8
D = 16
HD = H * D

NC = 2
NS = 16
NW = NC * NS
EPT = E // NW
CH = 200
CHH = 100
NCHUNK = EPT // CH
NP = 10240
RT = NP // NS
ZR = 128

_mesh = plsc.VectorSubcoreMesh(core_axis_name="c", subcore_axis_name="s")


def _tc1_body(fsrc_ref, fdst_ref, w_ref, wes_ref, wed_ref,
              fs_ref, es_ref, ed_ref, mes_ref, med_ref):
    i = pl.program_id(0)
    fs_ref[...] = jnp.dot(fsrc_ref[...], w_ref[...],
                          preferred_element_type=jnp.float32)
    es = jnp.dot(fsrc_ref[...], wes_ref[...],
                 preferred_element_type=jnp.float32)
    ed = jnp.dot(fdst_ref[...], wed_ref[...],
                 preferred_element_type=jnp.float32)
    es_ref[...] = es
    ed_ref[...] = ed
    mes = jnp.max(es, axis=0, keepdims=True)
    med = jnp.max(ed, axis=0, keepdims=True)

    @pl.when(i == 0)
    def _():
        mes_ref[...] = mes
        med_ref[...] = med

    @pl.when(i > 0)
    def _():
        mes_ref[...] = jnp.maximum(mes_ref[...], mes)
        med_ref[...] = jnp.maximum(med_ref[...], med)


def _tc1(feat_src, feat_dst, w_src, w_es16, w_ed16):
    blk = 1000
    grid = N // blk
    return pl.pallas_call(
        _tc1_body,
        grid=(grid,),
        in_specs=[
            pl.BlockSpec((blk, D_IN), lambda i: (i, 0)),
            pl.BlockSpec((blk, D_IN), lambda i: (i, 0)),
            pl.BlockSpec((D_IN, HD), lambda i: (0, 0)),
            pl.BlockSpec((D_IN, 16), lambda i: (0, 0)),
            pl.BlockSpec((D_IN, 16), lambda i: (0, 0)),
        ],
        out_specs=[
            pl.BlockSpec((blk, HD), lambda i: (i, 0)),
            pl.BlockSpec((blk, 16), lambda i: (i, 0)),
            pl.BlockSpec((blk, 16), lambda i: (i, 0)),
            pl.BlockSpec((1, 16), lambda i: (0, 0)),
            pl.BlockSpec((1, 16), lambda i: (0, 0)),
        ],
        out_shape=[
            jax.ShapeDtypeStruct((N, HD), jnp.float32),
            jax.ShapeDtypeStruct((N, 16), jnp.float32),
            jax.ShapeDtypeStruct((N, 16), jnp.float32),
            jax.ShapeDtypeStruct((1, 16), jnp.float32),
            jax.ShapeDtypeStruct((1, 16), jnp.float32),
        ],
    )(feat_src, feat_dst, w_src, w_es16, w_ed16)


@functools.partial(
    pl.kernel,
    out_type=[
        jax.ShapeDtypeStruct((E, 16), jnp.float32),
        jax.ShapeDtypeStruct((NC * NP, 16), jnp.float32),
    ],
    mesh=_mesh,
    compiler_params=pltpu.CompilerParams(use_tc_tiling_on_sc=False),
    scratch_types=[
        pltpu.VMEM((4, CHH), jnp.int32),
        pltpu.VMEM((CH, 16), jnp.float32),
        pltpu.VMEM((CH, 16), jnp.float32),
        pltpu.VMEM((CH, 16), jnp.float32),
        pltpu.VMEM((16,), jnp.float32),
        pltpu.VMEM((RT, 16), jnp.float32),
        pltpu.VMEM_SHARED((NP, 16), jnp.float32),
        pltpu.SemaphoreType.DMA,
        pltpu.SemaphoreType.DMA,
        pltpu.SemaphoreType.DMA,
        pltpu.SemaphoreType.DMA,
    ],
)
def _sc_a(ei2_h, es_h, ed_h, c_h, p_h, sp_h,
          sdb, esb, edb, pb, cb, zb, s_sh, sem1, sem2, sem3, sem4):
    cid = lax.axis_index("c")
    sid = lax.axis_index("s")
    wid = cid * NS + sid

    def _zrow(i, carry):
        zb[i] = jnp.zeros((16,), jnp.float32)
        return carry
    lax.fori_loop(0, RT, _zrow, 0)
    pltpu.sync_copy(zb, s_sh.at[pl.ds(sid * RT, RT)])
    plsc.subcore_barrier()

    pltpu.sync_copy(c_h, cb)
    cv = cb[...]

    base0 = wid * EPT

    def _chunk(j, carry):
        base = base0 + j * CH
        g = wid * NCHUNK + j
        pltpu.sync_copy(ei2_h.at[pl.ds(4 * g, 4)], sdb)
        d1 = pltpu.async_copy(es_h.at[sdb.at[0]], esb.at[pl.ds(0, CHH)], sem1)
        d2 = pltpu.async_copy(es_h.at[sdb.at[1]], esb.at[pl.ds(CHH, CHH)],
                              sem1)
        d3 = pltpu.async_copy(ed_h.at[sdb.at[2]], edb.at[pl.ds(0, CHH)], sem2)
        d4 = pltpu.async_copy(ed_h.at[sdb.at[3]], edb.at[pl.ds(CHH, CHH)],
                              sem2)
        d1.wait()
        d2.wait()
        d3.wait()
        d4.wait()

        U = 8

        def _inner(k, c2):
            for u in range(U):
                kk = k * U + u
                e = esb[kk] + edb[kk]
                e = jnp.where(e > 0, e, e * 0.01)
                pb[kk] = jnp.exp(e - cv)
            return c2
        lax.fori_loop(0, CH // U, _inner, 0)

        d5 = pltpu.async_copy(pb, p_h.at[pl.ds(base, CH)], sem3)
        d6 = pltpu.async_copy(pb.at[pl.ds(0, CHH)], s_sh.at[sdb.at[2]],
                              sem4, add=True)
        d7 = pltpu.async_copy(pb.at[pl.ds(CHH, CHH)], s_sh.at[sdb.at[3]],
                              sem4, add=True)
        d5.wait()
        d6.wait()
        d7.wait()
        return carry
    lax.fori_loop(0, NCHUNK, _chunk, 0)

    plsc.subcore_barrier()
    pltpu.sync_copy(s_sh.at[pl.ds(sid * RT, RT)],
                    sp_h.at[pl.ds(cid * NP + sid * RT, RT)])


def _tc2_body(sp_ref, rinv_ref):
    s = sp_ref[0:NP, :] + sp_ref[NP:2 * NP, :]
    rinv_ref[...] = 1.0 / s


def _tc2(sp):
    return pl.pallas_call(
        _tc2_body,
        out_shape=jax.ShapeDtypeStruct((NP, 16), jnp.float32),
    )(sp)


@functools.partial(
    pl.kernel,
    out_type=jax.ShapeDtypeStruct((NC * NP, HD), jnp.float32),
    mesh=_mesh,
    compiler_params=pltpu.CompilerParams(use_tc_tiling_on_sc=False),
    scratch_types=[
        pltpu.VMEM((4, CHH), jnp.int32),
        pltpu.VMEM((CH, HD), jnp.float32),
        pltpu.VMEM((CH, 16), jnp.float32),
        pltpu.VMEM((CH, 16), jnp.float32),
        pltpu.VMEM((ZR, HD), jnp.float32),
        pltpu.VMEM_SHARED((NP, HD), jnp.float32),
        pltpu.SemaphoreType.DMA,
        pltpu.SemaphoreType.DMA,
        pltpu.SemaphoreType.DMA,
    ],
)
def _sc_c(ei2_h, fs_h, rv_h, p_h, op_h,
          sdb, fsb, rvb, pb, zb, o_sh, sem1, sem2, sem3):
    cid = lax.axis_index("c")
    sid = lax.axis_index("s")
    wid = cid * NS + sid

    def _zrow(i, carry):
        for l in range(HD // 16):
            zb[i, pl.ds(l * 16, 16)] = jnp.zeros((16,), jnp.float32)
        return carry
    lax.fori_loop(0, ZR, _zrow, 0)
    for rep in range(RT // ZR):
        pltpu.sync_copy(zb, o_sh.at[pl.ds(sid * RT + rep * ZR, ZR)])
    plsc.subcore_barrier()

    base0 = wid * EPT

    def _chunk(j, carry):
        base = base0 + j * CH
        g = wid * NCHUNK + j
        pltpu.sync_copy(ei2_h.at[pl.ds(4 * g, 4)], sdb)
        d1 = pltpu.async_copy(fs_h.at[sdb.at[0]], fsb.at[pl.ds(0, CHH)], sem1)
        d2 = pltpu.async_copy(fs_h.at[sdb.at[1]], fsb.at[pl.ds(CHH, CHH)],
                              sem1)
        d3 = pltpu.async_copy(rv_h.at[sdb.at[2]], rvb.at[pl.ds(0, CHH)], sem2)
        d4 = pltpu.async_copy(rv_h.at[sdb.at[3]], rvb.at[pl.ds(CHH, CHH)],
                              sem2)
        d5 = pltpu.async_copy(p_h.at[pl.ds(base, CH)], pb, sem3)
        d1.wait()
        d2.wait()
        d3.wait()
        d4.wait()
        d5.wait()

        def _edge(k, c2):
            av = pb[k] * rvb[k]
            for h in range(H):
                a = av[h]
                fsb[k, pl.ds(h * D, D)] = fsb[k, pl.ds(h * D, D)] * a
            return c2
        lax.fori_loop(0, CH, _edge, 0)

        d6 = pltpu.async_copy(fsb.at[pl.ds(0, CHH)], o_sh.at[sdb.at[2]],
                              sem1, add=True)
        d7 = pltpu.async_copy(fsb.at[pl.ds(CHH, CHH)], o_sh.at[sdb.at[3]],
                              sem2, add=True)
        d6.wait()
        d7.wait()
        return carry
    lax.fori_loop(0, NCHUNK, _chunk, 0)

    plsc.subcore_barrier()
    for rep in range(RT // ZR):
        pltpu.sync_copy(o_sh.at[pl.ds(sid * RT + rep * ZR, ZR)],
                        op_h.at[pl.ds(cid * NP + sid * RT + rep * ZR, ZR)])


def _tc3_body(o0_ref, o1_ref, out_ref):
    out_ref[...] = jnp.maximum(o0_ref[...] + o1_ref[...], 0.0)


def _tc3(op):
    blk = 1024
    grid = NP // blk
    return pl.pallas_call(
        _tc3_body,
        grid=(grid,),
        in_specs=[
            pl.BlockSpec((blk, HD), lambda i: (i, 0)),
            pl.BlockSpec((blk, HD), lambda i: (NP // blk + i, 0)),
        ],
        out_specs=pl.BlockSpec((blk, HD), lambda i: (i, 0)),
        out_shape=jax.ShapeDtypeStruct((NP, HD), jnp.float32),
    )(op, op)


def kernel(feat_src, feat_dst, edge_index, dst_trans_weight, src_trans_weight,
           rel_emb, rel_trans_weight):
    ei = edge_index.astype(jnp.int32)
    src2 = ei[0].reshape(E // CH, 2, CHH)
    dst2 = ei[1].reshape(E // CH, 2, CHH)
    ei2 = jnp.concatenate([src2, dst2], axis=1).reshape(4 * (E // CH), CHH)

    rel_attn = jnp.matmul(rel_emb[None, :], rel_trans_weight).reshape(H, 2 * D)
    w_es = (src_trans_weight.reshape(D_IN, H, D)
            * rel_attn[:, D:][None]).sum(-1)
    w_ed = (dst_trans_weight.reshape(D_IN, H, D)
            * rel_attn[:, :D][None]).sum(-1)
    pad = jnp.zeros((D_IN, 16 - H), jnp.float32)
    w_es16 = jnp.concatenate([w_es, pad], axis=1)
    w_ed16 = jnp.concatenate([w_ed, pad], axis=1)

    fs, es16, ed16, mes, med = _tc1(feat_src, feat_dst, src_trans_weight,
                                    w_es16, w_ed16)
    c16 = jnp.maximum(mes + med, 0.0).reshape(16)

    p16, sp = _sc_a(ei2, es16, ed16, c16)
    rinv16 = _tc2(sp)
    op = _sc_c(ei2, fs, rinv16, p16)
    return _tc3(op)[:N]


# speedup vs baseline: 83.3487x; 1.0390x over previous
import functools

import jax
import jax.numpy as jnp
from jax import lax
from jax.experimental import pallas as pl
from jax.experimental.pallas import tpu as pltpu
from jax.experimental.pallas import tpu_sc as plsc

N = 10000
E = 320000
D_IN = 128
H = 8
D = 16
HD = H * D

NC = 2
NS = 16
NW = NC * NS
EPT = E // NW
CH = 200
CHH = 100
NCHUNK = EPT // CH
NP = 10240
RT = NP // NS
ZR = 128

_mesh = plsc.VectorSubcoreMesh(core_axis_name="c", subcore_axis_name="s")


def _tc1_body(fsrc_ref, fdst_ref, w_ref, wes_ref, wed_ref,
              fs_ref, es_ref, ed_ref, mes_ref, med_ref):
    i = pl.program_id(0)
    fs_ref[...] = jnp.dot(fsrc_ref[...], w_ref[...],
                          preferred_element_type=jnp.float32)
    es = jnp.dot(fsrc_ref[...], wes_ref[...],
                 preferred_element_type=jnp.float32)
    ed = jnp.dot(fdst_ref[...], wed_ref[...],
                 preferred_element_type=jnp.float32)
    es_ref[...] = es
    ed_ref[...] = ed
    mes = jnp.max(es, axis=0, keepdims=True)
    med = jnp.max(ed, axis=0, keepdims=True)

    @pl.when(i == 0)
    def _():
        mes_ref[...] = mes
        med_ref[...] = med

    @pl.when(i > 0)
    def _():
        mes_ref[...] = jnp.maximum(mes_ref[...], mes)
        med_ref[...] = jnp.maximum(med_ref[...], med)


def _tc1(feat_src, feat_dst, w_src, w_es16, w_ed16):
    blk = 1000
    grid = N // blk
    return pl.pallas_call(
        _tc1_body,
        grid=(grid,),
        in_specs=[
            pl.BlockSpec((blk, D_IN), lambda i: (i, 0)),
            pl.BlockSpec((blk, D_IN), lambda i: (i, 0)),
            pl.BlockSpec((D_IN, HD), lambda i: (0, 0)),
            pl.BlockSpec((D_IN, 16), lambda i: (0, 0)),
            pl.BlockSpec((D_IN, 16), lambda i: (0, 0)),
        ],
        out_specs=[
            pl.BlockSpec((blk, HD), lambda i: (i, 0)),
            pl.BlockSpec((blk, 16), lambda i: (i, 0)),
            pl.BlockSpec((blk, 16), lambda i: (i, 0)),
            pl.BlockSpec((1, 16), lambda i: (0, 0)),
            pl.BlockSpec((1, 16), lambda i: (0, 0)),
        ],
        out_shape=[
            jax.ShapeDtypeStruct((N, HD), jnp.float32),
            jax.ShapeDtypeStruct((N, 16), jnp.float32),
            jax.ShapeDtypeStruct((N, 16), jnp.float32),
            jax.ShapeDtypeStruct((1, 16), jnp.float32),
            jax.ShapeDtypeStruct((1, 16), jnp.float32),
        ],
    )(feat_src, feat_dst, w_src, w_es16, w_ed16)


@functools.partial(
    pl.kernel,
    out_type=[
        jax.ShapeDtypeStruct((E, 16), jnp.float32),
        jax.ShapeDtypeStruct((NC * NP, 16), jnp.float32),
    ],
    mesh=_mesh,
    compiler_params=pltpu.CompilerParams(use_tc_tiling_on_sc=False),
    scratch_types=[
        pltpu.VMEM((4 * NCHUNK, CHH), jnp.int32),
        pltpu.VMEM((CH, 16), jnp.float32),
        pltpu.VMEM((CH, 16), jnp.float32),
        pltpu.VMEM((CH, 16), jnp.float32),
        pltpu.VMEM((16,), jnp.float32),
        pltpu.VMEM((RT, 16), jnp.float32),
        pltpu.VMEM_SHARED((NP, 16), jnp.float32),
        pltpu.SemaphoreType.DMA,
        pltpu.SemaphoreType.DMA,
        pltpu.SemaphoreType.DMA,
        pltpu.SemaphoreType.DMA,
    ],
)
def _sc_a(ei2_h, es_h, ed_h, c_h, p_h, sp_h,
          sdb, esb, edb, pb, cb, zb, s_sh, sem1, sem2, sem3, sem4):
    cid = lax.axis_index("c")
    sid = lax.axis_index("s")
    wid = cid * NS + sid

    def _zrow(i, carry):
        zb[i] = jnp.zeros((16,), jnp.float32)
        return carry
    lax.fori_loop(0, RT, _zrow, 0)
    pltpu.sync_copy(zb, s_sh.at[pl.ds(sid * RT, RT)])
    plsc.subcore_barrier()

    pltpu.sync_copy(c_h, cb)
    cv = cb[...]

    base0 = wid * EPT
    pltpu.sync_copy(ei2_h.at[pl.ds(4 * wid * NCHUNK, 4 * NCHUNK)], sdb)

    def _chunk(j, carry):
        base = base0 + j * CH
        d1 = pltpu.async_copy(es_h.at[sdb.at[4 * j]], esb.at[pl.ds(0, CHH)],
                              sem1)
        d2 = pltpu.async_copy(es_h.at[sdb.at[4 * j + 1]],
                              esb.at[pl.ds(CHH, CHH)], sem1)
        d3 = pltpu.async_copy(ed_h.at[sdb.at[4 * j + 2]],
                              edb.at[pl.ds(0, CHH)], sem2)
        d4 = pltpu.async_copy(ed_h.at[sdb.at[4 * j + 3]],
                              edb.at[pl.ds(CHH, CHH)], sem2)
        d1.wait()
        d2.wait()
        d3.wait()
        d4.wait()

        U = 8

        def _inner(k, c2):
            for u in range(U):
                kk = k * U + u
                e = esb[kk] + edb[kk]
                e = jnp.where(e > 0, e, e * 0.01)
                pb[kk] = jnp.exp(e - cv)
            return c2
        lax.fori_loop(0, CH // U, _inner, 0)

        d5 = pltpu.async_copy(pb, p_h.at[pl.ds(base, CH)], sem3)
        d6 = pltpu.async_copy(pb.at[pl.ds(0, CHH)], s_sh.at[sdb.at[4 * j + 2]],
                              sem4, add=True)
        d7 = pltpu.async_copy(pb.at[pl.ds(CHH, CHH)],
                              s_sh.at[sdb.at[4 * j + 3]], sem4, add=True)
        d5.wait()
        d6.wait()
        d7.wait()
        return carry
    lax.fori_loop(0, NCHUNK, _chunk, 0)

    plsc.subcore_barrier()
    pltpu.sync_copy(s_sh.at[pl.ds(sid * RT, RT)],
                    sp_h.at[pl.ds(cid * NP + sid * RT, RT)])


def _tc2_body(sp_ref, rinv_ref):
    s = sp_ref[0:NP, :] + sp_ref[NP:2 * NP, :]
    rinv_ref[...] = 1.0 / s


def _tc2(sp):
    return pl.pallas_call(
        _tc2_body,
        out_shape=jax.ShapeDtypeStruct((NP, 16), jnp.float32),
    )(sp)


@functools.partial(
    pl.kernel,
    out_type=jax.ShapeDtypeStruct((NC * NP, HD), jnp.float32),
    mesh=_mesh,
    compiler_params=pltpu.CompilerParams(use_tc_tiling_on_sc=False),
    scratch_types=[
        pltpu.VMEM((4, CHH), jnp.int32),
        pltpu.VMEM((CH, HD), jnp.float32),
        pltpu.VMEM((CH, 16), jnp.float32),
        pltpu.VMEM((CH, 16), jnp.float32),
        pltpu.VMEM((ZR, HD), jnp.float32),
        pltpu.VMEM_SHARED((NP, HD), jnp.float32),
        pltpu.SemaphoreType.DMA,
        pltpu.SemaphoreType.DMA,
        pltpu.SemaphoreType.DMA,
    ],
)
def _sc_c(ei2_h, fs_h, rv_h, p_h, op_h,
          sdb, fsb, rvb, pb, zb, o_sh, sem1, sem2, sem3):
    cid = lax.axis_index("c")
    sid = lax.axis_index("s")
    wid = cid * NS + sid

    def _zrow(i, carry):
        for l in range(HD // 16):
            zb[i, pl.ds(l * 16, 16)] = jnp.zeros((16,), jnp.float32)
        return carry
    lax.fori_loop(0, ZR, _zrow, 0)
    for rep in range(RT // ZR):
        pltpu.sync_copy(zb, o_sh.at[pl.ds(sid * RT + rep * ZR, ZR)])
    plsc.subcore_barrier()

    base0 = wid * EPT

    def _chunk(j, carry):
        base = base0 + j * CH
        g = wid * NCHUNK + j
        pltpu.sync_copy(ei2_h.at[pl.ds(4 * g, 4)], sdb)
        d1 = pltpu.async_copy(fs_h.at[sdb.at[0]], fsb.at[pl.ds(0, CHH)], sem1)
        d2 = pltpu.async_copy(fs_h.at[sdb.at[1]], fsb.at[pl.ds(CHH, CHH)],
                              sem1)
        d3 = pltpu.async_copy(rv_h.at[sdb.at[2]], rvb.at[pl.ds(0, CHH)], sem2)
        d4 = pltpu.async_copy(rv_h.at[sdb.at[3]], rvb.at[pl.ds(CHH, CHH)],
                              sem2)
        d5 = pltpu.async_copy(p_h.at[pl.ds(base, CH)], pb, sem3)
        d1.wait()
        d2.wait()
        d3.wait()
        d4.wait()
        d5.wait()

        def _edge(k2, c2):
            for q in range(2):
                k = k2 * 2 + q
                av = pb[k] * rvb[k]
                for h in range(H):
                    a = av[h]
                    fsb[k, pl.ds(h * D, D)] = fsb[k, pl.ds(h * D, D)] * a
            return c2
        lax.fori_loop(0, CH // 2, _edge, 0)

        d6 = pltpu.async_copy(fsb.at[pl.ds(0, CHH)], o_sh.at[sdb.at[2]],
                              sem1, add=True)
        d7 = pltpu.async_copy(fsb.at[pl.ds(CHH, CHH)], o_sh.at[sdb.at[3]],
                              sem2, add=True)
        d6.wait()
        d7.wait()
        return carry
    lax.fori_loop(0, NCHUNK, _chunk, 0)

    plsc.subcore_barrier()
    for rep in range(RT // ZR):
        pltpu.sync_copy(o_sh.at[pl.ds(sid * RT + rep * ZR, ZR)],
                        op_h.at[pl.ds(cid * NP + sid * RT + rep * ZR, ZR)])


def _tc3_body(o0_ref, o1_ref, out_ref):
    out_ref[...] = jnp.maximum(o0_ref[...] + o1_ref[...], 0.0)


def _tc3(op):
    blk = 1024
    grid = NP // blk
    return pl.pallas_call(
        _tc3_body,
        grid=(grid,),
        in_specs=[
            pl.BlockSpec((blk, HD), lambda i: (i, 0)),
            pl.BlockSpec((blk, HD), lambda i: (NP // blk + i, 0)),
        ],
        out_specs=pl.BlockSpec((blk, HD), lambda i: (i, 0)),
        out_shape=jax.ShapeDtypeStruct((NP, HD), jnp.float32),
    )(op, op)


def kernel(feat_src, feat_dst, edge_index, dst_trans_weight, src_trans_weight,
           rel_emb, rel_trans_weight):
    ei = edge_index.astype(jnp.int32)
    src2 = ei[0].reshape(E // CH, 2, CHH)
    dst2 = ei[1].reshape(E // CH, 2, CHH)
    ei2 = jnp.concatenate([src2, dst2], axis=1).reshape(4 * (E // CH), CHH)

    rel_attn = jnp.matmul(rel_emb[None, :], rel_trans_weight).reshape(H, 2 * D)
    w_es = (src_trans_weight.reshape(D_IN, H, D)
            * rel_attn[:, D:][None]).sum(-1)
    w_ed = (dst_trans_weight.reshape(D_IN, H, D)
            * rel_attn[:, :D][None]).sum(-1)
    pad = jnp.zeros((D_IN, 16 - H), jnp.float32)
    w_es16 = jnp.concatenate([w_es, pad], axis=1)
    w_ed16 = jnp.concatenate([w_ed, pad], axis=1)

    fs, es16, ed16, mes, med = _tc1(feat_src, feat_dst, src_trans_weight,
                                    w_es16, w_ed16)
    c16 = jnp.maximum(mes + med, 0.0).reshape(16)

    p16, sp = _sc_a(ei2, es16, ed16, c16)
    rinv16 = _tc2(sp)
    op = _sc_c(ei2, fs, rinv16, p16)
    return _tc3(op)[:N]


# speedup vs baseline: 91.0755x; 1.0927x over previous
import functools

import jax
import jax.numpy as jnp
from jax import lax
from jax.experimental import pallas as pl
from jax.experimental.pallas import tpu as pltpu
from jax.experimental.pallas import tpu_sc as plsc

N = 10000
E = 320000
D_IN = 128
H = 8
D = 16
HD = H * D

NC = 2
NS = 16
NW = NC * NS
EPT = E // NW
CH = 200
CHH = 100
NCHUNK = EPT // CH
NP = 10240
RT = NP // NS
ZR = 128

_mesh = plsc.VectorSubcoreMesh(core_axis_name="c", subcore_axis_name="s")


def _tc1_body(fsrc_ref, fdst_ref, w_ref, wes_ref, wed_ref,
              fs_ref, es_ref, ed_ref, mes_ref, med_ref):
    i = pl.program_id(0)
    fs_ref[...] = jnp.dot(fsrc_ref[...], w_ref[...],
                          preferred_element_type=jnp.float32)
    es = jnp.dot(fsrc_ref[...], wes_ref[...],
                 preferred_element_type=jnp.float32)
    ed = jnp.dot(fdst_ref[...], wed_ref[...],
                 preferred_element_type=jnp.float32)
    es_ref[...] = es
    ed_ref[...] = ed
    mes = jnp.max(es, axis=0, keepdims=True)
    med = jnp.max(ed, axis=0, keepdims=True)

    @pl.when(i == 0)
    def _():
        mes_ref[...] = mes
        med_ref[...] = med

    @pl.when(i > 0)
    def _():
        mes_ref[...] = jnp.maximum(mes_ref[...], mes)
        med_ref[...] = jnp.maximum(med_ref[...], med)


def _tc1(feat_src, feat_dst, w_src, w_es16, w_ed16):
    blk = 1000
    grid = N // blk
    return pl.pallas_call(
        _tc1_body,
        grid=(grid,),
        in_specs=[
            pl.BlockSpec((blk, D_IN), lambda i: (i, 0)),
            pl.BlockSpec((blk, D_IN), lambda i: (i, 0)),
            pl.BlockSpec((D_IN, HD), lambda i: (0, 0)),
            pl.BlockSpec((D_IN, 16), lambda i: (0, 0)),
            pl.BlockSpec((D_IN, 16), lambda i: (0, 0)),
        ],
        out_specs=[
            pl.BlockSpec((blk, HD), lambda i: (i, 0)),
            pl.BlockSpec((blk, 16), lambda i: (i, 0)),
            pl.BlockSpec((blk, 16), lambda i: (i, 0)),
            pl.BlockSpec((1, 16), lambda i: (0, 0)),
            pl.BlockSpec((1, 16), lambda i: (0, 0)),
        ],
        out_shape=[
            jax.ShapeDtypeStruct((N, HD), jnp.float32),
            jax.ShapeDtypeStruct((N, 16), jnp.float32),
            jax.ShapeDtypeStruct((N, 16), jnp.float32),
            jax.ShapeDtypeStruct((1, 16), jnp.float32),
            jax.ShapeDtypeStruct((1, 16), jnp.float32),
        ],
    )(feat_src, feat_dst, w_src, w_es16, w_ed16)


@functools.partial(
    pl.kernel,
    out_type=[
        jax.ShapeDtypeStruct((E, 16), jnp.float32),
        jax.ShapeDtypeStruct((NC * NP, 16), jnp.float32),
    ],
    mesh=_mesh,
    compiler_params=pltpu.CompilerParams(use_tc_tiling_on_sc=False),
    scratch_types=[
        pltpu.VMEM((4 * NCHUNK, CHH), jnp.int32),
        pltpu.VMEM((CH, 16), jnp.float32),
        pltpu.VMEM((CH, 16), jnp.float32),
        pltpu.VMEM((CH, 16), jnp.float32),
        pltpu.VMEM((16,), jnp.float32),
        pltpu.VMEM((RT, 16), jnp.float32),
        pltpu.VMEM_SHARED((NP, 16), jnp.float32),
        pltpu.SemaphoreType.DMA,
        pltpu.SemaphoreType.DMA,
        pltpu.SemaphoreType.DMA,
        pltpu.SemaphoreType.DMA,
    ],
)
def _sc_a(ei2_h, es_h, ed_h, c_h, p_h, sp_h,
          sdb, esb, edb, pb, cb, zb, s_sh, sem1, sem2, sem3, sem4):
    cid = lax.axis_index("c")
    sid = lax.axis_index("s")
    wid = cid * NS + sid

    def _zrow(i, carry):
        zb[i] = jnp.zeros((16,), jnp.float32)
        return carry
    lax.fori_loop(0, RT, _zrow, 0)
    pltpu.sync_copy(zb, s_sh.at[pl.ds(sid * RT, RT)])
    plsc.subcore_barrier()

    pltpu.sync_copy(c_h, cb)
    cv = cb[...]

    base0 = wid * EPT
    pltpu.sync_copy(ei2_h.at[pl.ds(4 * wid * NCHUNK, 4 * NCHUNK)], sdb)

    def _chunk(j, carry):
        base = base0 + j * CH
        d1 = pltpu.async_copy(es_h.at[sdb.at[4 * j]], esb.at[pl.ds(0, CHH)],
                              sem1)
        d2 = pltpu.async_copy(es_h.at[sdb.at[4 * j + 1]],
                              esb.at[pl.ds(CHH, CHH)], sem1)
        d3 = pltpu.async_copy(ed_h.at[sdb.at[4 * j + 2]],
                              edb.at[pl.ds(0, CHH)], sem2)
        d4 = pltpu.async_copy(ed_h.at[sdb.at[4 * j + 3]],
                              edb.at[pl.ds(CHH, CHH)], sem2)
        d1.wait()
        d2.wait()
        d3.wait()
        d4.wait()

        U = 8

        def _inner(k, c2):
            for u in range(U):
                kk = k * U + u
                e = esb[kk] + edb[kk]
                e = jnp.where(e > 0, e, e * 0.01)
                pb[kk] = jnp.exp(e - cv)
            return c2
        lax.fori_loop(0, CH // U, _inner, 0)

        d5 = pltpu.async_copy(pb, p_h.at[pl.ds(base, CH)], sem3)
        d6 = pltpu.async_copy(pb.at[pl.ds(0, CHH)], s_sh.at[sdb.at[4 * j + 2]],
                              sem4, add=True)
        d7 = pltpu.async_copy(pb.at[pl.ds(CHH, CHH)],
                              s_sh.at[sdb.at[4 * j + 3]], sem4, add=True)
        d5.wait()
        d6.wait()
        d7.wait()
        return carry
    lax.fori_loop(0, NCHUNK, _chunk, 0)

    plsc.subcore_barrier()
    pltpu.sync_copy(s_sh.at[pl.ds(sid * RT, RT)],
                    sp_h.at[pl.ds(cid * NP + sid * RT, RT)])


def _tc2_body(sp_ref, rinv_ref):
    s = sp_ref[0:NP, :] + sp_ref[NP:2 * NP, :]
    rinv_ref[...] = 1.0 / s


def _tc2(sp):
    return pl.pallas_call(
        _tc2_body,
        out_shape=jax.ShapeDtypeStruct((NP, 16), jnp.float32),
    )(sp)


@functools.partial(
    pl.kernel,
    out_type=jax.ShapeDtypeStruct((NC * NP, HD), jnp.float32),
    mesh=_mesh,
    compiler_params=pltpu.CompilerParams(use_tc_tiling_on_sc=False),
    scratch_types=[
        pltpu.VMEM((4, CHH), jnp.int32),
        pltpu.VMEM((CH, HD), jnp.float32),
        pltpu.VMEM((CH, 16), jnp.float32),
        pltpu.VMEM((CH, 16), jnp.float32),
        pltpu.VMEM((ZR, HD), jnp.float32),
        pltpu.VMEM_SHARED((NP, HD), jnp.float32),
        pltpu.SemaphoreType.DMA,
        pltpu.SemaphoreType.DMA,
        pltpu.SemaphoreType.DMA,
    ],
)
def _sc_c(ei2_h, fs_h, rv_h, p_h, op_h,
          sdb, fsb, rvb, pb, zb, o_sh, sem1, sem2, sem3):
    cid = lax.axis_index("c")
    sid = lax.axis_index("s")
    wid = cid * NS + sid

    def _zrow(i, carry):
        for l in range(HD // 16):
            zb[i, pl.ds(l * 16, 16)] = jnp.zeros((16,), jnp.float32)
        return carry
    lax.fori_loop(0, ZR, _zrow, 0)
    for rep in range(RT // ZR):
        pltpu.sync_copy(zb, o_sh.at[pl.ds(sid * RT + rep * ZR, ZR)])
    plsc.subcore_barrier()

    base0 = wid * EPT

    def _chunk(j, carry):
        base = base0 + j * CH
        g = wid * NCHUNK + j
        pltpu.sync_copy(ei2_h.at[pl.ds(4 * g, 4)], sdb)
        d1 = pltpu.async_copy(fs_h.at[sdb.at[0]], fsb.at[pl.ds(0, CHH)], sem1)
        d2 = pltpu.async_copy(fs_h.at[sdb.at[1]], fsb.at[pl.ds(CHH, CHH)],
                              sem1)
        d3 = pltpu.async_copy(rv_h.at[sdb.at[2]], rvb.at[pl.ds(0, CHH)], sem2)
        d4 = pltpu.async_copy(rv_h.at[sdb.at[3]], rvb.at[pl.ds(CHH, CHH)],
                              sem2)
        d5 = pltpu.async_copy(p_h.at[pl.ds(base, CH)], pb, sem3)
        d3.wait()
        d4.wait()
        d5.wait()

        def _half(k0):
            def _edge(k2, c2):
                for q in range(2):
                    k = k0 + k2 * 2 + q
                    av = pb[k] * rvb[k]
                    for h in range(H):
                        a = av[h]
                        fsb[k, pl.ds(h * D, D)] = fsb[k, pl.ds(h * D, D)] * a
                return c2
            lax.fori_loop(0, CHH // 2, _edge, 0)

        d1.wait()
        _half(0)
        d6 = pltpu.async_copy(fsb.at[pl.ds(0, CHH)], o_sh.at[sdb.at[2]],
                              sem1, add=True)
        d2.wait()
        _half(CHH)
        d7 = pltpu.async_copy(fsb.at[pl.ds(CHH, CHH)], o_sh.at[sdb.at[3]],
                              sem2, add=True)
        d6.wait()
        d7.wait()
        return carry
    lax.fori_loop(0, NCHUNK, _chunk, 0)

    plsc.subcore_barrier()
    for rep in range(RT // ZR):
        pltpu.sync_copy(o_sh.at[pl.ds(sid * RT + rep * ZR, ZR)],
                        op_h.at[pl.ds(cid * NP + sid * RT + rep * ZR, ZR)])


def _tc3_body(o0_ref, o1_ref, out_ref):
    out_ref[...] = jnp.maximum(o0_ref[...] + o1_ref[...], 0.0)


def _tc3(op):
    blk = 1024
    grid = NP // blk
    return pl.pallas_call(
        _tc3_body,
        grid=(grid,),
        in_specs=[
            pl.BlockSpec((blk, HD), lambda i: (i, 0)),
            pl.BlockSpec((blk, HD), lambda i: (NP // blk + i, 0)),
        ],
        out_specs=pl.BlockSpec((blk, HD), lambda i: (i, 0)),
        out_shape=jax.ShapeDtypeStruct((NP, HD), jnp.float32),
    )(op, op)


def kernel(feat_src, feat_dst, edge_index, dst_trans_weight, src_trans_weight,
           rel_emb, rel_trans_weight):
    ei = edge_index.astype(jnp.int32)
    src2 = ei[0].reshape(E // CH, 2, CHH)
    dst2 = ei[1].reshape(E // CH, 2, CHH)
    ei2 = jnp.concatenate([src2, dst2], axis=1).reshape(4 * (E // CH), CHH)

    rel_attn = jnp.matmul(rel_emb[None, :], rel_trans_weight).reshape(H, 2 * D)
    w_es = (src_trans_weight.reshape(D_IN, H, D)
            * rel_attn[:, D:][None]).sum(-1)
    w_ed = (dst_trans_weight.reshape(D_IN, H, D)
            * rel_attn[:, :D][None]).sum(-1)
    pad = jnp.zeros((D_IN, 16 - H), jnp.float32)
    w_es16 = jnp.concatenate([w_es, pad], axis=1)
    w_ed16 = jnp.concatenate([w_ed, pad], axis=1)

    fs, es16, ed16, mes, med = _tc1(feat_src, feat_dst, src_trans_weight,
                                    w_es16, w_ed16)
    c16 = jnp.maximum(mes + med, 0.0).reshape(16)

    p16, sp = _sc_a(ei2, es16, ed16, c16)
    rinv16 = _tc2(sp)
    op = _sc_c(ei2, fs, rinv16, p16)
    return _tc3(op)[:N]
